# baseline (speedup 1.0000x reference)
import functools

import jax
import jax.numpy as jnp
from jax import lax
from jax.experimental import pallas as pl
from jax.experimental.pallas import tpu as pltpu
from jax.experimental.pallas import tpu_sc as plsc

_NC = 2
_NS = 16
_NW = _NC * _NS
_CHUNK = 80
_LANES = 16


def _proj_body(h_ref, w1_ref, b1_ref, w2_ref, q_ref):
    w2 = w2_ref[...]
    wc = jnp.dot(w2, w1_ref[...], preferred_element_type=jnp.float32)
    bc = jnp.dot(b1_ref[...], w2.T, preferred_element_type=jnp.float32)
    h16 = h_ref[...].astype(jnp.bfloat16)
    q_ref[...] = (
        jnp.dot(h16, wc.T.astype(jnp.bfloat16),
                preferred_element_type=jnp.float32)
        + bc
    )


def _project(hidden, w1, b1r, w2):
    n, h = hidden.shape
    s = w1.shape[0]
    br = 2000
    return pl.pallas_call(
        _proj_body,
        grid=(n // br,),
        in_specs=[
            pl.BlockSpec((br, h), lambda i: (i, 0)),
            pl.BlockSpec((s, h), lambda i: (0, 0)),
            pl.BlockSpec((1, s), lambda i: (0, 0)),
            pl.BlockSpec((s, s), lambda i: (0, 0)),
        ],
        out_specs=pl.BlockSpec((br, s), lambda i: (i, 0)),
        out_shape=jax.ShapeDtypeStruct((n, s), jnp.float32),
    )(hidden, w1, b1r, w2)


def _tr_body(i_ref, o_ref):
    o_ref[...] = i_ref[...].T


def _transpose_idx(idx_pad):
    npad, k = idx_pad.shape
    br = 2048
    return pl.pallas_call(
        _tr_body,
        grid=(npad // br,),
        in_specs=[pl.BlockSpec((br, k), lambda i: (i, 0))],
        out_specs=pl.BlockSpec((k, br), lambda i: (0, i)),
        out_shape=jax.ShapeDtypeStruct((k, npad), jnp.int32),
    )(idx_pad)


@functools.partial(jax.jit, static_argnums=(3, 4, 5, 6))
def _sc_pool_call(q, idx_prep, b2, n, npad, k, s):
    bw = npad // _NW
    nchunks = bw // _CHUNK
    rem = n - (_NW - 1) * bw
    mesh = plsc.VectorSubcoreMesh(core_axis_name="c", subcore_axis_name="s")
    inv = 1.0 / k

    @functools.partial(
        pl.kernel,
        out_type=jax.ShapeDtypeStruct((n, s), jnp.float32),
        mesh=mesh,
        scratch_types=[
            pltpu.VMEM((nchunks, k, _CHUNK), jnp.int32),
            pltpu.VMEM((bw, s), jnp.float32),
            pltpu.VMEM((bw, s), jnp.float32),
            pltpu.VMEM((bw, s), jnp.float32),
            pltpu.VMEM((s,), jnp.float32),
            pltpu.VMEM_SHARED((q.shape[0], s), jnp.float32),
            pltpu.SemaphoreType.DMA,
            pltpu.SemaphoreType.DMA,
        ],
        compiler_params=pltpu.CompilerParams(use_tc_tiling_on_sc=False),
    )
    def sc_pool(q_hbm, idx_hbm, b2_hbm, out_hbm, idx_v, acc_v,
                rows_a, rows_b, b2_v, q_sp, sem_a, sem_b):
        cid = lax.axis_index("c")
        sid = lax.axis_index("s")
        wid = sid * _NC + cid
        base = wid * bw

        @pl.when(sid == 0)
        def _():
            pltpu.sync_copy(q_hbm, q_sp)

        for j in range(nchunks):
            pltpu.sync_copy(
                idx_hbm.at[:, pl.ds(base + j * _CHUNK, _CHUNK)], idx_v.at[j]
            )
        pltpu.sync_copy(b2_hbm, b2_v)
        plsc.subcore_barrier()

        bufs = (rows_a, rows_b)
        sems = (sem_a, sem_b)

        def fire(kk, dst, sem):
            return [
                pltpu.async_copy(
                    q_sp.at[idx_v.at[j, kk]],
                    dst.at[pl.ds(j * _CHUNK, _CHUNK)],
                    sem,
                )
                for j in range(nchunks)
            ]

        def accum(src):
            @plsc.parallel_loop(0, bw, unroll=8)
            def _(a):
                for h2 in range(s // _LANES):
                    sl = pl.ds(h2 * _LANES, _LANES)
                    plsc.addupdate(acc_v.at[a, sl], src[a, sl])

        d0 = fire(0, acc_v, sem_a)
        d1 = fire(1, rows_b, sem_b)
        for d in d0:
            d.wait()
        prev = d1
        for kk in range(2, k + 1):
            cur = None
            if kk < k:
                cur = fire(kk, bufs[kk % 2], sems[kk % 2])
            for d in prev:
                d.wait()
            accum(bufs[(kk - 1) % 2])
            prev = cur

        @plsc.parallel_loop(0, bw, unroll=8)
        def _(a):
            for h2 in range(s // _LANES):
                sl = pl.ds(h2 * _LANES, _LANES)
                v = acc_v[a, sl] * inv + b2_v[sl]
                rows_a[a, sl] = jnp.maximum(v, 0.0)

        @pl.when(wid < _NW - 1)
        def _():
            pltpu.sync_copy(rows_a, out_hbm.at[pl.ds(base, bw)])

        @pl.when(wid == _NW - 1)
        def _():
            pltpu.sync_copy(
                rows_a.at[pl.ds(0, rem)],
                out_hbm.at[pl.ds((_NW - 1) * bw, rem)],
            )

    return sc_pool(q, idx_prep, b2)


def kernel(hidden_states, neighbor_indices, W1, b1, W2, b2):
    n, _ = hidden_states.shape
    k = neighbor_indices.shape[1]
    s = W1.shape[0]
    bw = -(-(-(-n // _NW)) // _CHUNK) * _CHUNK
    npad = bw * _NW

    q = _project(hidden_states, W1, b1.reshape(1, s), W2)

    idx = neighbor_indices.astype(jnp.int32)
    idx = jnp.pad(idx, ((0, npad - n), (0, 0)))
    idx_prep = _transpose_idx(idx)
    return _sc_pool_call(q, idx_prep, b2, n, npad, k, s)


# speedup vs baseline: 1.1912x; 1.1912x over previous
import functools

import jax
import jax.numpy as jnp
from jax import lax
from jax.experimental import pallas as pl
from jax.experimental.pallas import tpu as pltpu
from jax.experimental.pallas import tpu_sc as plsc

_NC = 2
_NS = 16
_NW = _NC * _NS
_CHUNK = 80
_LANES = 16


def _proj_body(h_ref, w1_ref, b1_ref, w2_ref, q_ref):
    w2 = w2_ref[...]
    wc = jnp.dot(w2, w1_ref[...], preferred_element_type=jnp.float32)
    bc = jnp.dot(b1_ref[...], w2.T, preferred_element_type=jnp.float32)
    h16 = h_ref[...].astype(jnp.bfloat16)
    q_ref[...] = (
        jnp.dot(h16, wc.T.astype(jnp.bfloat16),
                preferred_element_type=jnp.float32)
        + bc
    )


def _project(hidden, w1, b1r, w2):
    n, h = hidden.shape
    s = w1.shape[0]
    br = 2000
    return pl.pallas_call(
        _proj_body,
        grid=(n // br,),
        in_specs=[
            pl.BlockSpec((br, h), lambda i: (i, 0)),
            pl.BlockSpec((s, h), lambda i: (0, 0)),
            pl.BlockSpec((1, s), lambda i: (0, 0)),
            pl.BlockSpec((s, s), lambda i: (0, 0)),
        ],
        out_specs=pl.BlockSpec((br, s), lambda i: (i, 0)),
        out_shape=jax.ShapeDtypeStruct((n, s), jnp.float32),
    )(hidden, w1, b1r, w2)


@functools.partial(jax.jit, static_argnums=(3, 4, 5, 6))
def _sc_pool_call(q, idx_prep, b2, n, npad, k, s):
    bw = npad // _NW
    nchunks = bw // _CHUNK
    rem = n - (_NW - 1) * bw
    mesh = plsc.VectorSubcoreMesh(core_axis_name="c", subcore_axis_name="s")
    inv = 1.0 / k

    @functools.partial(
        pl.kernel,
        out_type=jax.ShapeDtypeStruct((n, s), jnp.float32),
        mesh=mesh,
        scratch_types=[
            pltpu.VMEM((k * nchunks, _CHUNK), jnp.int32),
            pltpu.VMEM((bw, s), jnp.float32),
            pltpu.VMEM((bw, s), jnp.float32),
            pltpu.VMEM((bw, s), jnp.float32),
            pltpu.VMEM((bw, s), jnp.float32),
            pltpu.VMEM((s,), jnp.float32),
            pltpu.VMEM_SHARED((q.shape[0], s), jnp.float32),
            pltpu.SemaphoreType.DMA,
            pltpu.SemaphoreType.DMA,
            pltpu.SemaphoreType.DMA,
            pltpu.SemaphoreType.DMA,
        ],
        compiler_params=pltpu.CompilerParams(use_tc_tiling_on_sc=False),
    )
    def sc_pool(q_hbm, idx_hbm, b2_hbm, out_hbm, idx_v, acc_v,
                rows_a, rows_b, rows_c, b2_v, q_sp,
                sem_0, sem_a, sem_b, sem_c):
        cid = lax.axis_index("c")
        sid = lax.axis_index("s")
        wid = sid * _NC + cid
        base = wid * bw

        @pl.when(sid == 0)
        def _():
            pltpu.sync_copy(q_hbm, q_sp)

        pltpu.sync_copy(idx_hbm.at[wid], idx_v)
        pltpu.sync_copy(b2_hbm, b2_v)
        plsc.subcore_barrier()

        bufs = (rows_a, rows_b, rows_c)
        sems = (sem_a, sem_b, sem_c)

        def fire(kk, dst, sem):
            return [
                pltpu.async_copy(
                    q_sp.at[idx_v.at[kk * nchunks + j]],
                    dst.at[pl.ds(j * _CHUNK, _CHUNK)],
                    sem,
                )
                for j in range(nchunks)
            ]

        def accum(src):
            @plsc.parallel_loop(0, bw, unroll=8)
            def _(a):
                for h2 in range(s // _LANES):
                    sl = pl.ds(h2 * _LANES, _LANES)
                    plsc.addupdate(acc_v.at[a, sl], src[a, sl])

        descs = {0: fire(0, acc_v, sem_0)}
        for kk in range(1, min(4, k)):
            descs[kk] = fire(kk, bufs[(kk - 1) % 3], sems[(kk - 1) % 3])
        for d in descs[0]:
            d.wait()
        for kk in range(1, k):
            for d in descs[kk]:
                d.wait()
            accum(bufs[(kk - 1) % 3])
            if kk + 3 < k:
                descs[kk + 3] = fire(
                    kk + 3, bufs[(kk + 2) % 3], sems[(kk + 2) % 3]
                )

        @plsc.parallel_loop(0, bw, unroll=8)
        def _(a):
            for h2 in range(s // _LANES):
                sl = pl.ds(h2 * _LANES, _LANES)
                v = acc_v[a, sl] * inv + b2_v[sl]
                rows_a[a, sl] = jnp.maximum(v, 0.0)

        @pl.when(wid < _NW - 1)
        def _():
            pltpu.sync_copy(rows_a, out_hbm.at[pl.ds(base, bw)])

        @pl.when(wid == _NW - 1)
        def _():
            pltpu.sync_copy(
                rows_a.at[pl.ds(0, rem)],
                out_hbm.at[pl.ds((_NW - 1) * bw, rem)],
            )

    return sc_pool(q, idx_prep, b2)


def kernel(hidden_states, neighbor_indices, W1, b1, W2, b2):
    n, _ = hidden_states.shape
    k = neighbor_indices.shape[1]
    s = W1.shape[0]
    bw = -(-(-(-n // _NW)) // _CHUNK) * _CHUNK
    npad = bw * _NW

    q = _project(hidden_states, W1, b1.reshape(1, s), W2)

    idx = neighbor_indices.astype(jnp.int32)
    idx = jnp.pad(idx, ((0, npad - n), (0, 0)))
    idx_prep = (
        idx.T.reshape(k, _NW, bw)
        .transpose(1, 0, 2)
        .reshape(_NW, k * (bw // _CHUNK), _CHUNK)
    )
    return _sc_pool_call(q, idx_prep, b2, n, npad, k, s)


# speedup vs baseline: 1.2103x; 1.0160x over previous
import functools

import jax
import jax.numpy as jnp
from jax import lax
from jax.experimental import pallas as pl
from jax.experimental.pallas import tpu as pltpu
from jax.experimental.pallas import tpu_sc as plsc

_NC = 2
_NS = 16
_NW = _NC * _NS
_CHUNK = 80
_LANES = 16


def _proj_body(h_ref, w1_ref, b1_ref, w2_ref, q_ref):
    w2 = w2_ref[...]
    wc = jnp.dot(w2, w1_ref[...], preferred_element_type=jnp.float32)
    bc = jnp.dot(b1_ref[...], w2.T, preferred_element_type=jnp.float32)
    h16 = h_ref[...].astype(jnp.bfloat16)
    q_ref[...] = (
        jnp.dot(h16, wc.T.astype(jnp.bfloat16),
                preferred_element_type=jnp.float32)
        + bc
    )


def _project(hidden, w1, b1r, w2):
    n, h = hidden.shape
    s = w1.shape[0]
    br = 2000
    return pl.pallas_call(
        _proj_body,
        grid=(n // br,),
        in_specs=[
            pl.BlockSpec((br, h), lambda i: (i, 0)),
            pl.BlockSpec((s, h), lambda i: (0, 0)),
            pl.BlockSpec((1, s), lambda i: (0, 0)),
            pl.BlockSpec((s, s), lambda i: (0, 0)),
        ],
        out_specs=pl.BlockSpec((br, s), lambda i: (i, 0)),
        out_shape=jax.ShapeDtypeStruct((n, s), jnp.float32),
    )(hidden, w1, b1r, w2)


@functools.partial(jax.jit, static_argnums=(3, 4, 5, 6))
def _sc_pool_call(q, idx_prep, b2, n, npad, k, s):
    bw = npad // _NW
    nchunks = bw // _CHUNK
    rem = n - (_NW - 1) * bw
    mesh = plsc.VectorSubcoreMesh(core_axis_name="c", subcore_axis_name="s")
    inv = 1.0 / k

    @functools.partial(
        pl.kernel,
        out_type=jax.ShapeDtypeStruct((n, s), jnp.float32),
        mesh=mesh,
        scratch_types=[
            pltpu.VMEM((k * nchunks, _CHUNK), jnp.int32),
            pltpu.VMEM((bw, s), jnp.float32),
            pltpu.VMEM((bw, s), jnp.float32),
            pltpu.VMEM((bw, s), jnp.float32),
            pltpu.VMEM((s,), jnp.float32),
            pltpu.VMEM_SHARED((q.shape[0], s), jnp.float32),
            pltpu.SemaphoreType.DMA,
            pltpu.SemaphoreType.DMA,
        ],
        compiler_params=pltpu.CompilerParams(use_tc_tiling_on_sc=False),
    )
    def sc_pool(q_hbm, idx_hbm, b2_hbm, out_hbm, idx_v, acc_v,
                rows_a, rows_b, b2_v, q_sp, sem_a, sem_b):
        cid = lax.axis_index("c")
        sid = lax.axis_index("s")
        wid = sid * _NC + cid
        base = wid * bw

        @pl.when(sid == 0)
        def _():
            pltpu.sync_copy(q_hbm, q_sp)

        pltpu.sync_copy(idx_hbm.at[wid], idx_v)
        pltpu.sync_copy(b2_hbm, b2_v)
        plsc.subcore_barrier()

        bufs = (rows_a, rows_b)
        sems = (sem_a, sem_b)

        def fire(kk, dst, sem):
            return [
                pltpu.async_copy(
                    q_sp.at[idx_v.at[kk * nchunks + j]],
                    dst.at[pl.ds(j * _CHUNK, _CHUNK)],
                    sem,
                )
                for j in range(nchunks)
            ]

        def accum(src):
            @plsc.parallel_loop(0, bw, unroll=8)
            def _(a):
                for h2 in range(s // _LANES):
                    sl = pl.ds(h2 * _LANES, _LANES)
                    plsc.addupdate(acc_v.at[a, sl], src[a, sl])

        d0 = fire(0, acc_v, sem_a)
        d1 = fire(1, rows_b, sem_b)
        for d in d0:
            d.wait()
        prev = d1
        for kk in range(2, k + 1):
            cur = None
            if kk < k:
                cur = fire(kk, bufs[kk % 2], sems[kk % 2])
            for d in prev:
                d.wait()
            accum(bufs[(kk - 1) % 2])
            prev = cur

        @plsc.parallel_loop(0, bw, unroll=8)
        def _(a):
            for h2 in range(s // _LANES):
                sl = pl.ds(h2 * _LANES, _LANES)
                v = acc_v[a, sl] * inv + b2_v[sl]
                rows_a[a, sl] = jnp.maximum(v, 0.0)

        @pl.when(wid < _NW - 1)
        def _():
            pltpu.sync_copy(rows_a, out_hbm.at[pl.ds(base, bw)])

        @pl.when(wid == _NW - 1)
        def _():
            pltpu.sync_copy(
                rows_a.at[pl.ds(0, rem)],
                out_hbm.at[pl.ds((_NW - 1) * bw, rem)],
            )

    return sc_pool(q, idx_prep, b2)


def kernel(hidden_states, neighbor_indices, W1, b1, W2, b2):
    n, _ = hidden_states.shape
    k = neighbor_indices.shape[1]
    s = W1.shape[0]
    bw = -(-(-(-n // _NW)) // _CHUNK) * _CHUNK
    npad = bw * _NW

    q = _project(hidden_states, W1, b1.reshape(1, s), W2)

    idx = neighbor_indices.astype(jnp.int32)
    idx = jnp.pad(idx, ((0, npad - n), (0, 0)))
    idx_prep = (
        idx.T.reshape(k, _NW, bw)
        .transpose(1, 0, 2)
        .reshape(_NW, k * (bw // _CHUNK), _CHUNK)
    )
    return _sc_pool_call(q, idx_prep, b2, n, npad, k, s)
